# baseline (speedup 1.0000x reference)
import functools

import jax
import jax.numpy as jnp
from jax.experimental import pallas as pl
from jax.experimental.pallas import tpu as pltpu


def _gru_cell_step(h, m, gi, whh, bhh, hid):
    h = h * m
    gh = jax.lax.dot_general(
        h.astype(jnp.bfloat16), whh,
        (((1,), (1,)), ((), ())),
        preferred_element_type=jnp.float32,
    ) + bhh
    r = jax.nn.sigmoid(gi[:, :hid] + gh[:, :hid])
    z = jax.nn.sigmoid(gi[:, hid:2 * hid] + gh[:, hid:2 * hid])
    n = jnp.tanh(gi[:, 2 * hid:] + r * gh[:, 2 * hid:])
    return (1.0 - z) * n + z * h


def _fused_body(x_ref, m0_ref, m1_ref,
                wih0_ref, whh0_ref, bih0_ref, bhh0_ref,
                wih1_ref, whh1_ref, bih1_ref, bhh1_ref,
                h00_ref, h10_ref,
                y_ref, hout0_ref, hout1_ref,
                h0_s, h1_s, y0_s, gi0_s, gi1_s,
                *, steps, n_envs, hid, nblk):
    i = pl.program_id(0)

    gi1_s[...] = jax.lax.dot_general(
        y0_s[...], wih1_ref[...],
        (((1,), (1,)), ((), ())),
        preferred_element_type=jnp.float32,
    ) + bih1_ref[...]

    gi0_s[...] = jax.lax.dot_general(
        x_ref[...], wih0_ref[...],
        (((1,), (1,)), ((), ())),
        preferred_element_type=jnp.float32,
    ) + bih0_ref[...]

    @pl.when(i == 0)
    def _():
        h0_s[...] = h00_ref[...]

    @pl.when(i == 1)
    def _():
        h1_s[...] = h10_ref[...]

    whh0 = whh0_ref[...]
    bhh0 = bhh0_ref[...]
    whh1 = whh1_ref[...]
    bhh1 = bhh1_ref[...]

    h0 = h0_s[...]
    h1 = h1_s[...]
    for b in range(steps):
        sl = slice(b * n_envs, (b + 1) * n_envs)
        h0 = _gru_cell_step(h0, m0_ref[b], gi0_s[sl, :], whh0, bhh0, hid)
        h1 = _gru_cell_step(h1, m1_ref[b], gi1_s[sl, :], whh1, bhh1, hid)
        y0_s[sl, :] = h0.astype(jnp.bfloat16)
        y_ref[sl, :] = h1
    h0_s[...] = h0
    h1_s[...] = h1

    @pl.when(i == nblk - 1)
    def _():
        hout0_ref[...] = h0

    hout1_ref[...] = h1


def kernel(x, hidden_states, masks, W_ih_l0, W_hh_l0, b_ih_l0, b_hh_l0,
           W_ih_l1, W_hh_l1, b_ih_l1, b_hh_l1):
    n_envs, n_layers, hid = hidden_states.shape
    t = x.shape[0] // n_envs
    d = x.shape[1]

    block_t = 16
    while t % block_t:
        block_t //= 2
    nblk = t // block_t
    bn = block_t * n_envs

    m3 = masks.reshape(t, n_envs, 1)
    bf = jnp.bfloat16

    body = functools.partial(
        _fused_body, steps=block_t, n_envs=n_envs, hid=hid, nblk=nblk)

    full = lambda i: (0, 0)
    y1, h0f, h1f = pl.pallas_call(
        body,
        grid=(nblk + 1,),
        in_specs=[
            pl.BlockSpec((bn, d), lambda i: (jnp.minimum(i, nblk - 1), 0)),
            pl.BlockSpec((block_t, n_envs, 1),
                         lambda i: (jnp.minimum(i, nblk - 1), 0, 0)),
            pl.BlockSpec((block_t, n_envs, 1),
                         lambda i: (jnp.maximum(i - 1, 0), 0, 0)),
            pl.BlockSpec((3 * hid, d), full),
            pl.BlockSpec((3 * hid, hid), full),
            pl.BlockSpec((1, 3 * hid), full),
            pl.BlockSpec((1, 3 * hid), full),
            pl.BlockSpec((3 * hid, hid), full),
            pl.BlockSpec((3 * hid, hid), full),
            pl.BlockSpec((1, 3 * hid), full),
            pl.BlockSpec((1, 3 * hid), full),
            pl.BlockSpec((n_envs, hid), full),
            pl.BlockSpec((n_envs, hid), full),
        ],
        out_specs=[
            pl.BlockSpec((bn, hid), lambda i: (jnp.maximum(i - 1, 0), 0)),
            pl.BlockSpec((n_envs, hid), full),
            pl.BlockSpec((n_envs, hid), full),
        ],
        out_shape=[
            jax.ShapeDtypeStruct((t * n_envs, hid), jnp.float32),
            jax.ShapeDtypeStruct((n_envs, hid), jnp.float32),
            jax.ShapeDtypeStruct((n_envs, hid), jnp.float32),
        ],
        scratch_shapes=[
            pltpu.VMEM((n_envs, hid), jnp.float32),
            pltpu.VMEM((n_envs, hid), jnp.float32),
            pltpu.VMEM((bn, hid), jnp.bfloat16),
            pltpu.VMEM((bn, 3 * hid), jnp.float32),
            pltpu.VMEM((bn, 3 * hid), jnp.float32),
        ],
        compiler_params=pltpu.CompilerParams(
            dimension_semantics=("arbitrary",),
        ),
    )(x.astype(bf), m3, m3,
      W_ih_l0.astype(bf), W_hh_l0.astype(bf),
      b_ih_l0.reshape(1, -1), b_hh_l0.reshape(1, -1),
      W_ih_l1.astype(bf), W_hh_l1.astype(bf),
      b_ih_l1.reshape(1, -1), b_hh_l1.reshape(1, -1),
      hidden_states[:, 0, :], hidden_states[:, 1, :])

    hidden_out = jnp.stack([h0f, h1f], axis=1)
    return y1, hidden_out
